# baseline (speedup 1.0000x reference)
import functools

import jax
import jax.numpy as jnp
from jax import lax
from jax.experimental import pallas as pl
from jax.experimental.pallas import tpu as pltpu
from jax.experimental.pallas import tpu_sc as plsc

N = 50000
E = 800000
H = 64

NS = 16
NC = 2
NW = NC * NS
PER_TILE = 1568
SPLIT = NS * PER_TILE
NPAD = 2 * SPLIT
DUMMY = SPLIT
ACC_ROWS = SPLIT + 8

CK = 128
SUB = 3
BLK_E = CK * SUB

EPW = E // NW
CHUNKS = [2000] * 12 + [1000]
FLUSH = 2000
OBUF = 4096
CAP = 27648


def _sc_partition(src2, dst2):
    mesh = plsc.VectorSubcoreMesh(core_axis_name="c", subcore_axis_name="s")

    @functools.partial(
        pl.kernel,
        out_type=(
            jax.ShapeDtypeStruct((NC, NW, CAP), jnp.int32),
            jax.ShapeDtypeStruct((NC, NW, CAP), jnp.int32),
            jax.ShapeDtypeStruct((NW, 16), jnp.int32),
        ),
        mesh=mesh,
        scratch_types=[
            pltpu.VMEM((max(CHUNKS),), jnp.int32),
            pltpu.VMEM((max(CHUNKS),), jnp.int32),
            pltpu.VMEM((OBUF,), jnp.int32),
            pltpu.VMEM((OBUF,), jnp.int32),
            pltpu.VMEM((OBUF,), jnp.int32),
            pltpu.VMEM((OBUF,), jnp.int32),
            pltpu.VMEM((16,), jnp.int32),
        ],
        compiler_params=pltpu.CompilerParams(use_tc_tiling_on_sc=False,
                                             needs_layout_passes=False),
    )
    def part(src_hbm, dst_hbm, psrc_hbm, pdl_hbm, cnt_hbm,
             sin, din, os0, od0, os1, od1, cbuf):
        c = lax.axis_index("c")
        s = lax.axis_index("s")
        w = s * NC + c
        iota = lax.iota(jnp.int32, 16)

        def flush_if(os_b, od_b, off, fl, c2):
            do = off >= FLUSH

            @pl.when(do)
            def _():
                fla = pl.multiple_of(fl, 8)
                pltpu.sync_copy(os_b.at[pl.ds(0, FLUSH)],
                                psrc_hbm.at[c2, w, pl.ds(fla, FLUSH)])
                pltpu.sync_copy(od_b.at[pl.ds(0, FLUSH)],
                                pdl_hbm.at[c2, w, pl.ds(fla, FLUSH)])
                ngt = (off - FLUSH + 15) // 16

                def sh(i, carry):
                    os_b[pl.ds(i * 16, 16)] = os_b[pl.ds(FLUSH + i * 16, 16)]
                    od_b[pl.ds(i * 16, 16)] = od_b[pl.ds(FLUSH + i * 16, 16)]
                    return carry

                lax.fori_loop(0, ngt, sh, 0)

            return (jnp.where(do, off - FLUSH, off),
                    jnp.where(do, fl + FLUSH, fl))

        def compact(r, d, m_valid, nvalid, o0, o1):
            m0 = m_valid & (d < SPLIT)
            m1 = m_valid & (d >= SPLIT)
            cs0 = jnp.cumsum(m0.astype(jnp.int32))
            cs1 = jnp.cumsum(m1.astype(jnp.int32))
            idx0 = o0 + cs0 - 1
            idx1 = o1 + cs1 - 1
            plsc.store_scatter(os0, [idx0], r, mask=m0)
            plsc.store_scatter(od0, [idx0], d, mask=m0)
            plsc.store_scatter(os1, [idx1], r, mask=m1)
            plsc.store_scatter(od1, [idx1], d - SPLIT, mask=m1)
            return o0 + jnp.max(cs0), o1 + jnp.max(cs1)

        off0 = jnp.int32(0)
        off1 = jnp.int32(0)
        fl0 = jnp.int32(0)
        fl1 = jnp.int32(0)
        pos = 0
        for ce in CHUNKS:
            pltpu.sync_copy(src_hbm.at[w, pl.ds(pos, ce)], sin.at[pl.ds(0, ce)])
            pltpu.sync_copy(dst_hbm.at[w, pl.ds(pos, ce)], din.at[pl.ds(0, ce)])
            ng = ce // 16

            def grp(g, carry):
                o0, o1 = carry
                d = din[pl.ds(g * 16, 16)]
                r = sin[pl.ds(g * 16, 16)]
                return compact(r, d, iota >= 0, 16, o0, o1)

            off0, off1 = lax.fori_loop(0, ng, grp, (off0, off1))
            rem = ce % 16
            if rem:
                d = din[pl.ds(ce - 16, 16)]
                r = sin[pl.ds(ce - 16, 16)]
                off0, off1 = compact(r, d, iota >= (16 - rem), rem, off0, off1)
            off0, fl0 = flush_if(os0, od0, off0, fl0, 0)
            off1, fl1 = flush_if(os1, od1, off1, fl1, 1)
            pos += ce

        zsrc = jnp.zeros((16,), jnp.int32)
        zdl = jnp.full((16,), DUMMY, jnp.int32)
        for k in range(BLK_E // 16):
            i0 = off0 + k * 16 + iota
            i1 = off1 + k * 16 + iota
            plsc.store_scatter(os0, [i0], zsrc)
            plsc.store_scatter(od0, [i0], zdl)
            plsc.store_scatter(os1, [i1], zsrc)
            plsc.store_scatter(od1, [i1], zdl)
        fl0a = pl.multiple_of(fl0, 8)
        fl1a = pl.multiple_of(fl1, 8)
        pltpu.sync_copy(os0.at[pl.ds(0, FLUSH + BLK_E)],
                        psrc_hbm.at[0, w, pl.ds(fl0a, FLUSH + BLK_E)])
        pltpu.sync_copy(od0.at[pl.ds(0, FLUSH + BLK_E)],
                        pdl_hbm.at[0, w, pl.ds(fl0a, FLUSH + BLK_E)])
        pltpu.sync_copy(os1.at[pl.ds(0, FLUSH + BLK_E)],
                        psrc_hbm.at[1, w, pl.ds(fl1a, FLUSH + BLK_E)])
        pltpu.sync_copy(od1.at[pl.ds(0, FLUSH + BLK_E)],
                        pdl_hbm.at[1, w, pl.ds(fl1a, FLUSH + BLK_E)])

        real0 = fl0 + off0
        real1 = fl1 + off1
        nb0 = (real0 + BLK_E - 1) // BLK_E
        nb1 = (real1 + BLK_E - 1) // BLK_E
        cbuf[...] = jnp.where(iota == 0, nb0, jnp.where(iota == 1, nb1, 0))
        pltpu.sync_copy(cbuf, cnt_hbm.at[w])

    return part(src2, dst2)


def _sc_segment_sum(h, psrc, pdl4, cnt, zrows):
    mesh = plsc.VectorSubcoreMesh(core_axis_name="c", subcore_axis_name="s")

    @functools.partial(
        pl.kernel,
        out_type=jax.ShapeDtypeStruct((NPAD, H), jnp.float32),
        mesh=mesh,
        scratch_types=[
            pltpu.VMEM_SHARED((ACC_ROWS, H), jnp.float32),
            pltpu.VMEM((BLK_E,), jnp.int32),
            pltpu.VMEM((SUB, CK), jnp.int32),
            pltpu.VMEM((16,), jnp.int32),
            [pltpu.VMEM((CK, H), jnp.float32) for _ in range(SUB)],
            [pltpu.SemaphoreType.DMA for _ in range(SUB)],
            [pltpu.SemaphoreType.DMA for _ in range(SUB)],
        ],
        compiler_params=pltpu.CompilerParams(use_tc_tiling_on_sc=False,
                                             needs_layout_passes=False),
    )
    def seg(h_hbm, psrc_hbm, pdl_hbm, cnt_hbm, z_hbm, agg_hbm,
            acc, srcbuf, dlbuf, cbuf, stages, semG, semS):
        c = lax.axis_index("c")
        s = lax.axis_index("s")
        iota = lax.iota(jnp.int32, 16)

        pltpu.sync_copy(z_hbm, acc.at[pl.ds(s * PER_TILE, PER_TILE)])

        @pl.when(s == 0)
        def _():
            pltpu.sync_copy(z_hbm.at[pl.ds(0, 8)], acc.at[pl.ds(SPLIT, 8)])

        plsc.subcore_barrier()

        for k in range(2):
            widx = 2 * s + k
            pltpu.sync_copy(cnt_hbm.at[widx], cbuf)
            nb = jnp.max(jnp.where(iota == c, cbuf[...], 0))

            def blk(b, carry):
                pltpu.sync_copy(
                    psrc_hbm.at[c, widx, pl.ds(pl.multiple_of(b * BLK_E, 8),
                                               BLK_E)], srcbuf)
                pltpu.sync_copy(pdl_hbm.at[c, widx, pl.ds(SUB * b, SUB)], dlbuf)
                gd = [pltpu.async_copy(
                          h_hbm.at[srcbuf.at[pl.ds(j * CK, CK)]],
                          stages[j], semG[j]) for j in range(SUB)]
                sd = []
                for j in range(SUB):
                    gd[j].wait()
                    sd.append(pltpu.async_copy(stages[j], acc.at[dlbuf.at[j]],
                                               semS[j], add=True))
                for j in range(SUB):
                    sd[j].wait()
                return carry

            lax.fori_loop(0, nb, blk, 0)

        plsc.subcore_barrier()

        obase = c * SPLIT + s * PER_TILE
        pltpu.sync_copy(acc.at[pl.ds(s * PER_TILE, PER_TILE)],
                        agg_hbm.at[pl.ds(obase, PER_TILE)])

    return seg(h, psrc, pdl4, cnt, zrows)


def _tc_pre(x_p, W_pre_p, b_pre):
    rb = 2000

    def body(x_ref, w_ref, b_ref, o_ref):
        t = jnp.dot(x_ref[...], w_ref[...], preferred_element_type=jnp.float32)
        o_ref[...] = jnp.maximum(t + b_ref[...], 0.0)

    return pl.pallas_call(
        body,
        grid=(N // rb,),
        in_specs=[
            pl.BlockSpec((rb, 8), lambda i: (i, 0)),
            pl.BlockSpec((8, H), lambda i: (0, 0)),
            pl.BlockSpec((1, H), lambda i: (0, 0)),
        ],
        out_specs=pl.BlockSpec((rb, H), lambda i: (i, 0)),
        out_shape=jax.ShapeDtypeStruct((N, H), jnp.float32),
    )(x_p, W_pre_p, b_pre)


def _tc_layer(agg, h, W_rel, b_rel, W_root):
    rb = 2000

    def body(a_ref, h_ref, wr_ref, b_ref, wo_ref, o_ref):
        t = jnp.dot(a_ref[...], wr_ref[...], preferred_element_type=jnp.float32)
        t = t + jnp.dot(h_ref[...], wo_ref[...], preferred_element_type=jnp.float32)
        o_ref[...] = jnp.maximum(t + b_ref[...], 0.0)

    return pl.pallas_call(
        body,
        grid=(N // rb,),
        in_specs=[
            pl.BlockSpec((rb, H), lambda i: (i, 0)),
            pl.BlockSpec((rb, H), lambda i: (i, 0)),
            pl.BlockSpec((H, H), lambda i: (0, 0)),
            pl.BlockSpec((1, H), lambda i: (0, 0)),
            pl.BlockSpec((H, H), lambda i: (0, 0)),
        ],
        out_specs=pl.BlockSpec((rb, H), lambda i: (i, 0)),
        out_shape=jax.ShapeDtypeStruct((N, H), jnp.float32),
    )(agg, h, W_rel, b_rel, W_root)


def _tc_layer_post(agg, h, W_rel, b_rel, W_root, W_post, b_post):
    rb = 2000

    def body(a_ref, h_ref, wr_ref, b_ref, wo_ref, wp_ref, bp_ref, o_ref):
        t = jnp.dot(a_ref[...], wr_ref[...], preferred_element_type=jnp.float32)
        t = t + jnp.dot(h_ref[...], wo_ref[...], preferred_element_type=jnp.float32)
        t = jnp.maximum(t + b_ref[...], 0.0)
        u = jnp.dot(t, wp_ref[...], preferred_element_type=jnp.float32)
        o_ref[...] = jnp.maximum(u + bp_ref[...], 0.0)

    return pl.pallas_call(
        body,
        grid=(N // rb,),
        in_specs=[
            pl.BlockSpec((rb, H), lambda i: (i, 0)),
            pl.BlockSpec((rb, H), lambda i: (i, 0)),
            pl.BlockSpec((H, H), lambda i: (0, 0)),
            pl.BlockSpec((1, H), lambda i: (0, 0)),
            pl.BlockSpec((H, H), lambda i: (0, 0)),
            pl.BlockSpec((H, 2), lambda i: (0, 0)),
            pl.BlockSpec((1, 2), lambda i: (0, 0)),
        ],
        out_specs=pl.BlockSpec((rb, 2), lambda i: (i, 0)),
        out_shape=jax.ShapeDtypeStruct((N, 2), jnp.float32),
    )(agg, h, W_rel, b_rel, W_root, W_post, b_post)


def kernel(x, edge_index, W_pre, b_pre, W_rel0, b_rel0, W_root0,
           W_rel1, b_rel1, W_root1, W_post, b_post):
    src = edge_index[0]
    dst = edge_index[1]

    psrc, pdl, cnt = _sc_partition(src.reshape(NW, EPW), dst.reshape(NW, EPW))
    pdl4 = pdl.reshape(NC, NW, CAP // CK, CK)

    x_p = jnp.pad(x, ((0, 0), (0, 8 - x.shape[1])))
    W_pre_p = jnp.pad(W_pre, ((0, 8 - W_pre.shape[0]), (0, 0)))

    zrows = jnp.zeros((PER_TILE, H), jnp.float32)

    h0 = _tc_pre(x_p, W_pre_p, b_pre.reshape(1, H))
    agg0 = _sc_segment_sum(h0, psrc, pdl4, cnt, zrows)
    h1 = _tc_layer(agg0, h0, W_rel0, b_rel0.reshape(1, H), W_root0)
    agg1 = _sc_segment_sum(h1, psrc, pdl4, cnt, zrows)
    return _tc_layer_post(agg1, h1, W_rel1, b_rel1.reshape(1, H), W_root1,
                          W_post, b_post.reshape(1, 2))


# speedup vs baseline: 1.0071x; 1.0071x over previous
import functools

import jax
import jax.numpy as jnp
from jax import lax
from jax.experimental import pallas as pl
from jax.experimental.pallas import tpu as pltpu
from jax.experimental.pallas import tpu_sc as plsc

N = 50000
E = 800000
H = 64

NS = 16
NC = 2
NW = NC * NS
PER_TILE = 1568
SPLIT = NS * PER_TILE
NPAD = 2 * SPLIT
DUMMY = SPLIT
ACC_ROWS = SPLIT + 8

CK = 64
SUB = 6
BLK_E = CK * SUB

EPW = E // NW
CHUNKS = [2000] * 12 + [1000]
FLUSH = 2000
OBUF = 4096
CAP = 27648


def _sc_partition(src2, dst2):
    mesh = plsc.VectorSubcoreMesh(core_axis_name="c", subcore_axis_name="s")

    @functools.partial(
        pl.kernel,
        out_type=(
            jax.ShapeDtypeStruct((NC, NW, CAP), jnp.int32),
            jax.ShapeDtypeStruct((NC, NW, CAP), jnp.int32),
            jax.ShapeDtypeStruct((NW, 16), jnp.int32),
        ),
        mesh=mesh,
        scratch_types=[
            pltpu.VMEM((max(CHUNKS),), jnp.int32),
            pltpu.VMEM((max(CHUNKS),), jnp.int32),
            pltpu.VMEM((OBUF,), jnp.int32),
            pltpu.VMEM((OBUF,), jnp.int32),
            pltpu.VMEM((OBUF,), jnp.int32),
            pltpu.VMEM((OBUF,), jnp.int32),
            pltpu.VMEM((16,), jnp.int32),
        ],
        compiler_params=pltpu.CompilerParams(use_tc_tiling_on_sc=False,
                                             needs_layout_passes=False),
    )
    def part(src_hbm, dst_hbm, psrc_hbm, pdl_hbm, cnt_hbm,
             sin, din, os0, od0, os1, od1, cbuf):
        c = lax.axis_index("c")
        s = lax.axis_index("s")
        w = s * NC + c
        iota = lax.iota(jnp.int32, 16)

        def flush_if(os_b, od_b, off, fl, c2):
            do = off >= FLUSH

            @pl.when(do)
            def _():
                fla = pl.multiple_of(fl, 8)
                pltpu.sync_copy(os_b.at[pl.ds(0, FLUSH)],
                                psrc_hbm.at[c2, w, pl.ds(fla, FLUSH)])
                pltpu.sync_copy(od_b.at[pl.ds(0, FLUSH)],
                                pdl_hbm.at[c2, w, pl.ds(fla, FLUSH)])
                ngt = (off - FLUSH + 15) // 16

                def sh(i, carry):
                    os_b[pl.ds(i * 16, 16)] = os_b[pl.ds(FLUSH + i * 16, 16)]
                    od_b[pl.ds(i * 16, 16)] = od_b[pl.ds(FLUSH + i * 16, 16)]
                    return carry

                lax.fori_loop(0, ngt, sh, 0)

            return (jnp.where(do, off - FLUSH, off),
                    jnp.where(do, fl + FLUSH, fl))

        def compact(r, d, m_valid, nvalid, o0, o1):
            m0 = m_valid & (d < SPLIT)
            m1 = m_valid & (d >= SPLIT)
            cs0 = jnp.cumsum(m0.astype(jnp.int32))
            cs1 = jnp.cumsum(m1.astype(jnp.int32))
            idx0 = o0 + cs0 - 1
            idx1 = o1 + cs1 - 1
            plsc.store_scatter(os0, [idx0], r, mask=m0)
            plsc.store_scatter(od0, [idx0], d, mask=m0)
            plsc.store_scatter(os1, [idx1], r, mask=m1)
            plsc.store_scatter(od1, [idx1], d - SPLIT, mask=m1)
            return o0 + jnp.max(cs0), o1 + jnp.max(cs1)

        off0 = jnp.int32(0)
        off1 = jnp.int32(0)
        fl0 = jnp.int32(0)
        fl1 = jnp.int32(0)
        pos = 0
        for ce in CHUNKS:
            pltpu.sync_copy(src_hbm.at[w, pl.ds(pos, ce)], sin.at[pl.ds(0, ce)])
            pltpu.sync_copy(dst_hbm.at[w, pl.ds(pos, ce)], din.at[pl.ds(0, ce)])
            ng = ce // 16

            def grp(g, carry):
                o0, o1 = carry
                d = din[pl.ds(g * 16, 16)]
                r = sin[pl.ds(g * 16, 16)]
                return compact(r, d, iota >= 0, 16, o0, o1)

            off0, off1 = lax.fori_loop(0, ng, grp, (off0, off1))
            rem = ce % 16
            if rem:
                d = din[pl.ds(ce - 16, 16)]
                r = sin[pl.ds(ce - 16, 16)]
                off0, off1 = compact(r, d, iota >= (16 - rem), rem, off0, off1)
            off0, fl0 = flush_if(os0, od0, off0, fl0, 0)
            off1, fl1 = flush_if(os1, od1, off1, fl1, 1)
            pos += ce

        zsrc = jnp.zeros((16,), jnp.int32)
        zdl = jnp.full((16,), DUMMY, jnp.int32)
        for k in range(BLK_E // 16):
            i0 = off0 + k * 16 + iota
            i1 = off1 + k * 16 + iota
            plsc.store_scatter(os0, [i0], zsrc)
            plsc.store_scatter(od0, [i0], zdl)
            plsc.store_scatter(os1, [i1], zsrc)
            plsc.store_scatter(od1, [i1], zdl)
        fl0a = pl.multiple_of(fl0, 8)
        fl1a = pl.multiple_of(fl1, 8)
        pltpu.sync_copy(os0.at[pl.ds(0, FLUSH + BLK_E)],
                        psrc_hbm.at[0, w, pl.ds(fl0a, FLUSH + BLK_E)])
        pltpu.sync_copy(od0.at[pl.ds(0, FLUSH + BLK_E)],
                        pdl_hbm.at[0, w, pl.ds(fl0a, FLUSH + BLK_E)])
        pltpu.sync_copy(os1.at[pl.ds(0, FLUSH + BLK_E)],
                        psrc_hbm.at[1, w, pl.ds(fl1a, FLUSH + BLK_E)])
        pltpu.sync_copy(od1.at[pl.ds(0, FLUSH + BLK_E)],
                        pdl_hbm.at[1, w, pl.ds(fl1a, FLUSH + BLK_E)])

        real0 = fl0 + off0
        real1 = fl1 + off1
        nb0 = (real0 + BLK_E - 1) // BLK_E
        nb1 = (real1 + BLK_E - 1) // BLK_E
        cbuf[...] = jnp.where(iota == 0, nb0, jnp.where(iota == 1, nb1, 0))
        pltpu.sync_copy(cbuf, cnt_hbm.at[w])

    return part(src2, dst2)


def _sc_segment_sum(h, psrc, pdl4, cnt, zrows):
    mesh = plsc.VectorSubcoreMesh(core_axis_name="c", subcore_axis_name="s")

    @functools.partial(
        pl.kernel,
        out_type=jax.ShapeDtypeStruct((NPAD, H), jnp.float32),
        mesh=mesh,
        scratch_types=[
            pltpu.VMEM_SHARED((ACC_ROWS, H), jnp.float32),
            pltpu.VMEM((BLK_E,), jnp.int32),
            pltpu.VMEM((SUB, CK), jnp.int32),
            pltpu.VMEM((16,), jnp.int32),
            [pltpu.VMEM((CK, H), jnp.float32) for _ in range(SUB)],
            [pltpu.SemaphoreType.DMA for _ in range(SUB)],
            [pltpu.SemaphoreType.DMA for _ in range(SUB)],
        ],
        compiler_params=pltpu.CompilerParams(use_tc_tiling_on_sc=False,
                                             needs_layout_passes=False),
    )
    def seg(h_hbm, psrc_hbm, pdl_hbm, cnt_hbm, z_hbm, agg_hbm,
            acc, srcbuf, dlbuf, cbuf, stages, semG, semS):
        c = lax.axis_index("c")
        s = lax.axis_index("s")
        iota = lax.iota(jnp.int32, 16)

        pltpu.sync_copy(z_hbm, acc.at[pl.ds(s * PER_TILE, PER_TILE)])

        @pl.when(s == 0)
        def _():
            pltpu.sync_copy(z_hbm.at[pl.ds(0, 8)], acc.at[pl.ds(SPLIT, 8)])

        plsc.subcore_barrier()

        for k in range(2):
            widx = 2 * s + k
            pltpu.sync_copy(cnt_hbm.at[widx], cbuf)
            nb = jnp.max(jnp.where(iota == c, cbuf[...], 0))

            def blk(b, carry):
                pltpu.sync_copy(
                    psrc_hbm.at[c, widx, pl.ds(pl.multiple_of(b * BLK_E, 8),
                                               BLK_E)], srcbuf)
                pltpu.sync_copy(pdl_hbm.at[c, widx, pl.ds(SUB * b, SUB)], dlbuf)
                gd = [pltpu.async_copy(
                          h_hbm.at[srcbuf.at[pl.ds(j * CK, CK)]],
                          stages[j], semG[j]) for j in range(SUB)]
                sd = []
                for j in range(SUB):
                    gd[j].wait()
                    sd.append(pltpu.async_copy(stages[j], acc.at[dlbuf.at[j]],
                                               semS[j], add=True))
                for j in range(SUB):
                    sd[j].wait()
                return carry

            lax.fori_loop(0, nb, blk, 0)

        plsc.subcore_barrier()

        obase = c * SPLIT + s * PER_TILE
        pltpu.sync_copy(acc.at[pl.ds(s * PER_TILE, PER_TILE)],
                        agg_hbm.at[pl.ds(obase, PER_TILE)])

    return seg(h, psrc, pdl4, cnt, zrows)


def _tc_pre(x_p, W_pre_p, b_pre):
    rb = 2000

    def body(x_ref, w_ref, b_ref, o_ref):
        t = jnp.dot(x_ref[...], w_ref[...], preferred_element_type=jnp.float32)
        o_ref[...] = jnp.maximum(t + b_ref[...], 0.0)

    return pl.pallas_call(
        body,
        grid=(N // rb,),
        in_specs=[
            pl.BlockSpec((rb, 8), lambda i: (i, 0)),
            pl.BlockSpec((8, H), lambda i: (0, 0)),
            pl.BlockSpec((1, H), lambda i: (0, 0)),
        ],
        out_specs=pl.BlockSpec((rb, H), lambda i: (i, 0)),
        out_shape=jax.ShapeDtypeStruct((N, H), jnp.float32),
    )(x_p, W_pre_p, b_pre)


def _tc_layer(agg, h, W_rel, b_rel, W_root):
    rb = 2000

    def body(a_ref, h_ref, wr_ref, b_ref, wo_ref, o_ref):
        t = jnp.dot(a_ref[...], wr_ref[...], preferred_element_type=jnp.float32)
        t = t + jnp.dot(h_ref[...], wo_ref[...], preferred_element_type=jnp.float32)
        o_ref[...] = jnp.maximum(t + b_ref[...], 0.0)

    return pl.pallas_call(
        body,
        grid=(N // rb,),
        in_specs=[
            pl.BlockSpec((rb, H), lambda i: (i, 0)),
            pl.BlockSpec((rb, H), lambda i: (i, 0)),
            pl.BlockSpec((H, H), lambda i: (0, 0)),
            pl.BlockSpec((1, H), lambda i: (0, 0)),
            pl.BlockSpec((H, H), lambda i: (0, 0)),
        ],
        out_specs=pl.BlockSpec((rb, H), lambda i: (i, 0)),
        out_shape=jax.ShapeDtypeStruct((N, H), jnp.float32),
    )(agg, h, W_rel, b_rel, W_root)


def _tc_layer_post(agg, h, W_rel, b_rel, W_root, W_post, b_post):
    rb = 2000

    def body(a_ref, h_ref, wr_ref, b_ref, wo_ref, wp_ref, bp_ref, o_ref):
        t = jnp.dot(a_ref[...], wr_ref[...], preferred_element_type=jnp.float32)
        t = t + jnp.dot(h_ref[...], wo_ref[...], preferred_element_type=jnp.float32)
        t = jnp.maximum(t + b_ref[...], 0.0)
        u = jnp.dot(t, wp_ref[...], preferred_element_type=jnp.float32)
        o_ref[...] = jnp.maximum(u + bp_ref[...], 0.0)

    return pl.pallas_call(
        body,
        grid=(N // rb,),
        in_specs=[
            pl.BlockSpec((rb, H), lambda i: (i, 0)),
            pl.BlockSpec((rb, H), lambda i: (i, 0)),
            pl.BlockSpec((H, H), lambda i: (0, 0)),
            pl.BlockSpec((1, H), lambda i: (0, 0)),
            pl.BlockSpec((H, H), lambda i: (0, 0)),
            pl.BlockSpec((H, 2), lambda i: (0, 0)),
            pl.BlockSpec((1, 2), lambda i: (0, 0)),
        ],
        out_specs=pl.BlockSpec((rb, 2), lambda i: (i, 0)),
        out_shape=jax.ShapeDtypeStruct((N, 2), jnp.float32),
    )(agg, h, W_rel, b_rel, W_root, W_post, b_post)


def kernel(x, edge_index, W_pre, b_pre, W_rel0, b_rel0, W_root0,
           W_rel1, b_rel1, W_root1, W_post, b_post):
    src = edge_index[0]
    dst = edge_index[1]

    psrc, pdl, cnt = _sc_partition(src.reshape(NW, EPW), dst.reshape(NW, EPW))
    pdl4 = pdl.reshape(NC, NW, CAP // CK, CK)

    x_p = jnp.pad(x, ((0, 0), (0, 8 - x.shape[1])))
    W_pre_p = jnp.pad(W_pre, ((0, 8 - W_pre.shape[0]), (0, 0)))

    zrows = jnp.zeros((PER_TILE, H), jnp.float32)

    h0 = _tc_pre(x_p, W_pre_p, b_pre.reshape(1, H))
    agg0 = _sc_segment_sum(h0, psrc, pdl4, cnt, zrows)
    h1 = _tc_layer(agg0, h0, W_rel0, b_rel0.reshape(1, H), W_root0)
    agg1 = _sc_segment_sum(h1, psrc, pdl4, cnt, zrows)
    return _tc_layer_post(agg1, h1, W_rel1, b_rel1.reshape(1, H), W_root1,
                          W_post, b_post.reshape(1, 2))
